# Initial kernel scaffold; baseline (speedup 1.0000x reference)
import jax
import jax.numpy as jnp
from jax.experimental import pallas as pl


def kernel(x, edge_index, edge_attr, E1, E2, W1, b1, W2, b2):
    raise NotImplementedError("write your pallas kernel here")


# speedup vs baseline: 1.7400x; 1.7400x over previous
import functools

import jax
import jax.numpy as jnp
from jax import lax
from jax.experimental import pallas as pl
from jax.experimental.pallas import tpu as pltpu
from jax.experimental.pallas import tpu_sc as plsc

N, E, D, H = 10000, 160000, 256, 512
QD = 64
NQ = 4
NC = 2
NT = 16
EPT = E // NT
CH = 80
NCHUNK = EPT // CH
NP = 10240
RPT = NP // NT


def _sc_aggregate(xall, srcs4, eidx4, dsts, za):
    mesh = plsc.VectorSubcoreMesh(core_axis_name="c", subcore_axis_name="s")

    @functools.partial(
        pl.kernel,
        mesh=mesh,
        compiler_params=pltpu.CompilerParams(use_tc_tiling_on_sc=False),
        out_type=jax.ShapeDtypeStruct((NQ, NP, QD), jnp.float32),
        scratch_types=[
            pltpu.VMEM((NCHUNK, CH), jnp.int32),
            pltpu.VMEM((NCHUNK, CH), jnp.int32),
            pltpu.VMEM((NCHUNK, CH), jnp.int32),
            pltpu.VMEM((CH, QD), jnp.float32),
            pltpu.VMEM((CH, QD), jnp.float32),
            pltpu.VMEM_SHARED((NP, QD), jnp.float32),
            pltpu.SemaphoreType.DMA,
            pltpu.SemaphoreType.DMA,
        ],
    )
    def k(xall_h, srcs_h, eidx_h, dsts_h, za_h, aggr_o,
          src_v, eidx_v, dst_v, rows_v, rows2_v, aggr_s, sem, sem2):
        c = lax.axis_index("c")
        s = lax.axis_index("s")
        pltpu.sync_copy(dsts_h.at[s], dst_v)

        for q in range(2):
            qi = 2 * q + c
            pltpu.sync_copy(srcs_h.at[qi, s], src_v)
            pltpu.sync_copy(eidx_h.at[qi, s], eidx_v)
            pltpu.sync_copy(za_h, aggr_s.at[pl.ds(s * RPT, RPT)])
            plsc.subcore_barrier()

            def body(j, carry):
                cpx = pltpu.async_copy(xall_h.at[src_v.at[j]], rows_v, sem)
                cpe = pltpu.async_copy(xall_h.at[eidx_v.at[j]], rows2_v, sem2)
                cpx.wait()
                pltpu.sync_copy(rows_v, aggr_s.at[dst_v.at[j]], add=True)
                cpe.wait()
                pltpu.sync_copy(rows2_v, aggr_s.at[dst_v.at[j]], add=True)
                return carry

            lax.fori_loop(0, NCHUNK, body, 0)
            plsc.subcore_barrier()
            pltpu.sync_copy(aggr_s.at[pl.ds(s * RPT, RPT)],
                            aggr_o.at[qi, pl.ds(s * RPT, RPT)])

    return k(xall, srcs4, eidx4, dsts, za)


def _tc_mlp(aggr4, x, cconst, W1, b1, W2, b2):
    R = 400
    G = N // R

    def body(a4_ref, x_ref, cc_ref, w1_ref, b1_ref, w2_ref, b2_ref, o_ref):
        a = jnp.concatenate(
            [a4_ref[0], a4_ref[1], a4_ref[2], a4_ref[3]], axis=1)
        a = a + x_ref[...] + cc_ref[...]
        h1 = jnp.dot(a, w1_ref[...], preferred_element_type=jnp.float32)
        h1 = jnp.maximum(h1 + b1_ref[...], 0.0)
        o_ref[...] = jnp.dot(h1, w2_ref[...],
                             preferred_element_type=jnp.float32) + b2_ref[...]

    return pl.pallas_call(
        body,
        grid=(G,),
        in_specs=[
            pl.BlockSpec((NQ, R, QD), lambda i: (0, i, 0)),
            pl.BlockSpec((R, D), lambda i: (i, 0)),
            pl.BlockSpec((1, D), lambda i: (0, 0)),
            pl.BlockSpec((D, H), lambda i: (0, 0)),
            pl.BlockSpec((1, H), lambda i: (0, 0)),
            pl.BlockSpec((H, D), lambda i: (0, 0)),
            pl.BlockSpec((1, D), lambda i: (0, 0)),
        ],
        out_specs=pl.BlockSpec((R, D), lambda i: (i, 0)),
        out_shape=jax.ShapeDtypeStruct((N, D), jnp.float32),
    )(aggr4, x, cconst, W1, b1, W2, b2)


def kernel(x, edge_index, edge_attr, E1, E2, W1, b1, W2, b2):
    src = edge_index[0].astype(jnp.int32)
    dst = edge_index[1].astype(jnp.int32)
    combo = (edge_attr[:, 0] * 3 + edge_attr[:, 1]).astype(jnp.int32)
    k9 = jnp.arange(9)
    embC = (E1[k9 // 3] + E2[k9 % 3]).astype(jnp.float32)
    xq = jnp.concatenate([x[:, k * QD:(k + 1) * QD] for k in range(NQ)],
                         axis=0)
    eq = jnp.concatenate([embC[:, k * QD:(k + 1) * QD] for k in range(NQ)],
                         axis=0)
    xall = jnp.concatenate([xq, eq], axis=0)
    srcs4 = jnp.stack([src + k * N for k in range(NQ)]).reshape(
        NQ, NT, NCHUNK, CH)
    eidx4 = jnp.stack([NQ * N + k * 9 + combo for k in range(NQ)]).reshape(
        NQ, NT, NCHUNK, CH)
    dsts = dst.reshape(NT, NCHUNK, CH)
    za = jnp.zeros((RPT, QD), jnp.float32)
    cconst = (E1[4] + E2[0]).reshape(1, D)
    aggr4 = _sc_aggregate(xall, srcs4, eidx4, dsts, za)
    return _tc_mlp(aggr4, x, cconst, W1, b1.reshape(1, H), W2,
                   b2.reshape(1, D))
